# baseline (speedup 1.0000x reference)
import functools

import jax
import jax.numpy as jnp
from jax import lax
from jax.experimental import pallas as pl
from jax.experimental.pallas import tpu as pltpu
from jax.experimental.pallas import tpu_sc as plsc

GS = 256
NP = 2_000_000
NC = 2
NS = 16
NW = NC * NS
NB = NP // 128
BLOCKS = 25
CHUNK = BLOCKS * 128
NCHUNK = NP // CHUNK
BASE_TRIPS = NCHUNK // NW
EXTRA = NCHUNK % NW


def _deposit_body(pos_hbm, chg_hbm, out_hbm, grid_v, pos_v, chg_v, sems):
    c = lax.axis_index("c")
    s = lax.axis_index("s")
    wid = s * NC + c
    zeros16 = jnp.zeros((16,), jnp.float32)

    def zero_row(r, carry):
        for g in range(GS // 16):
            grid_v[r, pl.ds(g * 16, 16)] = zeros16
        return carry

    lax.fori_loop(0, GS, zero_row, 0)

    trips = jnp.where(wid < EXTRA, BASE_TRIPS + 1, BASE_TRIPS)

    def start_fetch(j, b):
        ck = wid + j * NW
        pltpu.async_copy(pos_hbm.at[pl.ds(ck * BLOCKS, BLOCKS)],
                         pos_v.at[b], sems[b])
        pltpu.async_copy(chg_hbm.at[pl.ds(ck * CHUNK, CHUNK)],
                         chg_v.at[b], sems[b])

    def wait_fetch(j, b):
        ck = wid + j * NW
        pltpu.make_async_copy(pos_hbm.at[pl.ds(ck * BLOCKS, BLOCKS)],
                              pos_v.at[b], sems[b]).wait()
        pltpu.make_async_copy(chg_hbm.at[pl.ds(ck * CHUNK, CHUNK)],
                              chg_v.at[b], sems[b]).wait()

    def compute_chunk(b):
        @plsc.parallel_loop(0, BLOCKS, unroll=25)
        def block_body(blk):
            for sub in range(8):
                xs = pos_v[b, blk, 0, pl.ds(sub * 16, 16)]
                ys = pos_v[b, blk, 1, pl.ds(sub * 16, 16)]
                cs = chg_v[b, pl.ds(blk * 128 + sub * 16, 16)]
                cxi = xs.astype(jnp.int32)
                cyi = ys.astype(jnp.int32)
                fx = xs - cxi.astype(jnp.float32)
                fy = ys - cyi.astype(jnp.float32)
                cx = jnp.minimum(cxi, GS - 2)
                cy = jnp.minimum(cyi, GS - 2)
                cx1 = cx + 1
                cy1 = cy + 1
                gx = 1.0 - fx
                gy = 1.0 - fy
                a = gx * cs
                b2 = fx * cs
                plsc.addupdate_scatter(grid_v, [cx, cy], a * gy)
                plsc.addupdate_scatter(grid_v, [cx1, cy], b2 * gy)
                plsc.addupdate_scatter(grid_v, [cx, cy1], a * fy)
                plsc.addupdate_scatter(grid_v, [cx1, cy1], b2 * fy)

    start_fetch(0, 0)

    def pair_body(p, carry):
        for b in range(2):
            j = p * 2 + b

            @pl.when(j < trips)
            def _():
                wait_fetch(j, b)

                @pl.when(j + 1 < trips)
                def _():
                    start_fetch(j + 1, 1 - b)

                compute_chunk(b)

        return carry

    lax.fori_loop(0, (BASE_TRIPS + 2) // 2, pair_body, 0)

    pltpu.sync_copy(grid_v, out_hbm.at[wid])


_deposit = functools.partial(
    pl.kernel,
    out_type=jax.ShapeDtypeStruct((NW, GS, GS), jnp.float32),
    mesh=plsc.VectorSubcoreMesh(core_axis_name="c", subcore_axis_name="s"),
    scratch_types=[
        pltpu.VMEM((GS, GS), jnp.float32),
        pltpu.VMEM((2, BLOCKS, 2, 128), jnp.float32),
        pltpu.VMEM((2, CHUNK), jnp.float32),
        [pltpu.SemaphoreType.DMA, pltpu.SemaphoreType.DMA],
    ],
    compiler_params=pltpu.CompilerParams(needs_layout_passes=False),
)(_deposit_body)


def _sum_partials_body(p_ref, o_ref):
    o_ref[...] = jnp.sum(p_ref[...], axis=0)


_sum_partials = pl.pallas_call(
    _sum_partials_body,
    out_shape=jax.ShapeDtypeStruct((GS, GS), jnp.float32),
)


def kernel(positions, charges):
    pos_blocked = positions.T.reshape(2, NB, 128).transpose(1, 0, 2)
    partials = _deposit(pos_blocked, charges)
    return _sum_partials(partials)


# speedup vs baseline: 1.0110x; 1.0110x over previous
import functools

import jax
import jax.numpy as jnp
from jax import lax
from jax.experimental import pallas as pl
from jax.experimental.pallas import tpu as pltpu
from jax.experimental.pallas import tpu_sc as plsc

GS = 256
NP = 2_000_000
NC = 2
NS = 16
NW = NC * NS
NB = NP // 128
BLOCKS = 25
CHUNK = BLOCKS * 128
NCHUNK = NP // CHUNK
BASE_TRIPS = NCHUNK // NW
EXTRA = NCHUNK % NW


def _deposit_body(pos_hbm, chg_hbm, out_hbm, grid_v, pos_v, chg_v, sems):
    c = lax.axis_index("c")
    s = lax.axis_index("s")
    wid = s * NC + c
    zeros16 = jnp.zeros((16,), jnp.float32)

    def zero_row(r, carry):
        for g in range(GS // 16):
            grid_v[r, pl.ds(g * 16, 16)] = zeros16
        return carry

    lax.fori_loop(0, GS, zero_row, 0)

    trips = jnp.where(wid < EXTRA, BASE_TRIPS + 1, BASE_TRIPS)

    def start_fetch(j, b):
        ck = wid + j * NW
        pltpu.async_copy(pos_hbm.at[pl.ds(ck * BLOCKS, BLOCKS)],
                         pos_v.at[b], sems[b])
        pltpu.async_copy(chg_hbm.at[pl.ds(ck * CHUNK, CHUNK)],
                         chg_v.at[b], sems[b])

    def wait_fetch(j, b):
        ck = wid + j * NW
        pltpu.make_async_copy(pos_hbm.at[pl.ds(ck * BLOCKS, BLOCKS)],
                              pos_v.at[b], sems[b]).wait()
        pltpu.make_async_copy(chg_hbm.at[pl.ds(ck * CHUNK, CHUNK)],
                              chg_v.at[b], sems[b]).wait()

    def compute_chunk(b):
        @plsc.parallel_loop(0, BLOCKS, unroll=10)
        def block_body(blk):
            for sub in range(8):
                xs = pos_v[b, blk, 0, pl.ds(sub * 16, 16)]
                ys = pos_v[b, blk, 1, pl.ds(sub * 16, 16)]
                cs = chg_v[b, pl.ds(blk * 128 + sub * 16, 16)]
                cxi = xs.astype(jnp.int32)
                cyi = ys.astype(jnp.int32)
                fx = xs - cxi.astype(jnp.float32)
                fy = ys - cyi.astype(jnp.float32)
                cx = jnp.minimum(cxi, GS - 2)
                cy = jnp.minimum(cyi, GS - 2)
                cx1 = cx + 1
                cy1 = cy + 1
                gx = 1.0 - fx
                gy = 1.0 - fy
                a = gx * cs
                b2 = fx * cs
                plsc.addupdate_scatter(grid_v, [cx, cy], a * gy)
                plsc.addupdate_scatter(grid_v, [cx1, cy], b2 * gy)
                plsc.addupdate_scatter(grid_v, [cx, cy1], a * fy)
                plsc.addupdate_scatter(grid_v, [cx1, cy1], b2 * fy)

    start_fetch(0, 0)

    def pair_body(p, carry):
        for b in range(2):
            j = p * 2 + b

            @pl.when(j < trips)
            def _():
                wait_fetch(j, b)

                @pl.when(j + 1 < trips)
                def _():
                    start_fetch(j + 1, 1 - b)

                compute_chunk(b)

        return carry

    lax.fori_loop(0, (BASE_TRIPS + 2) // 2, pair_body, 0)

    pltpu.sync_copy(grid_v, out_hbm.at[wid])


_deposit = functools.partial(
    pl.kernel,
    out_type=jax.ShapeDtypeStruct((NW, GS, GS), jnp.float32),
    mesh=plsc.VectorSubcoreMesh(core_axis_name="c", subcore_axis_name="s"),
    scratch_types=[
        pltpu.VMEM((GS, GS), jnp.float32),
        pltpu.VMEM((2, BLOCKS, 2, 128), jnp.float32),
        pltpu.VMEM((2, CHUNK), jnp.float32),
        [pltpu.SemaphoreType.DMA, pltpu.SemaphoreType.DMA],
    ],
    compiler_params=pltpu.CompilerParams(needs_layout_passes=False),
)(_deposit_body)


def _sum_partials_body(p_ref, o_ref):
    o_ref[...] = jnp.sum(p_ref[...], axis=0)


_sum_partials = pl.pallas_call(
    _sum_partials_body,
    out_shape=jax.ShapeDtypeStruct((GS, GS), jnp.float32),
)


def kernel(positions, charges):
    pos_blocked = positions.T.reshape(2, NB, 128).transpose(1, 0, 2)
    partials = _deposit(pos_blocked, charges)
    return _sum_partials(partials)


# speedup vs baseline: 2.3181x; 2.2929x over previous
import functools

import jax
import jax.numpy as jnp
from jax import lax
from jax.experimental import pallas as pl
from jax.experimental.pallas import tpu as pltpu
from jax.experimental.pallas import tpu_sc as plsc

GS = 256
NP = 2_000_000
NC = 2
NS = 16
NW = NC * NS
NB = NP // 128
BLOCKS = 25
CHUNK = BLOCKS * 128
NCHUNK = NP // CHUNK
BASE_TRIPS = NCHUNK // NW
EXTRA = NCHUNK % NW


def _deposit_body(pos_hbm, chg_hbm, out_hbm, grid_v, pos_v, chg_v, sems):
    c = lax.axis_index("c")
    s = lax.axis_index("s")
    wid = s * NC + c
    zeros16 = jnp.zeros((16,), jnp.float32)

    @plsc.parallel_loop(0, GS * GS // 16, unroll=8)
    def zero_body(i):
        grid_v[pl.ds(i * 16, 16)] = zeros16

    trips = jnp.where(wid < EXTRA, BASE_TRIPS + 1, BASE_TRIPS)

    def start_fetch(j, b):
        ck = wid + j * NW
        pltpu.async_copy(pos_hbm.at[pl.ds(ck * BLOCKS, BLOCKS)],
                         pos_v.at[b], sems[b])
        pltpu.async_copy(chg_hbm.at[pl.ds(ck * CHUNK, CHUNK)],
                         chg_v.at[b], sems[b])

    def wait_fetch(j, b):
        ck = wid + j * NW
        pltpu.make_async_copy(pos_hbm.at[pl.ds(ck * BLOCKS, BLOCKS)],
                              pos_v.at[b], sems[b]).wait()
        pltpu.make_async_copy(chg_hbm.at[pl.ds(ck * CHUNK, CHUNK)],
                              chg_v.at[b], sems[b]).wait()

    def compute_chunk(b):
        @plsc.parallel_loop(0, BLOCKS, unroll=5)
        def block_body(blk):
            for sub in range(8):
                xs = pos_v[b, blk, 0, pl.ds(sub * 16, 16)]
                ys = pos_v[b, blk, 1, pl.ds(sub * 16, 16)]
                cs = chg_v[b, pl.ds(blk * 128 + sub * 16, 16)]
                cxi = xs.astype(jnp.int32)
                cyi = ys.astype(jnp.int32)
                fx = xs - cxi.astype(jnp.float32)
                fy = ys - cyi.astype(jnp.float32)
                cx = jnp.minimum(cxi, GS - 2)
                cy = jnp.minimum(cyi, GS - 2)
                flat = lax.shift_left(cx, 8) + cy
                flat01 = flat + 1
                flat10 = flat + GS
                flat11 = flat + (GS + 1)
                gx = 1.0 - fx
                gy = 1.0 - fy
                a = gx * cs
                b2 = fx * cs
                plsc.addupdate_scatter(grid_v, [flat], a * gy)
                plsc.addupdate_scatter(grid_v, [flat10], b2 * gy)
                plsc.addupdate_scatter(grid_v, [flat01], a * fy)
                plsc.addupdate_scatter(grid_v, [flat11], b2 * fy)

    start_fetch(0, 0)

    def pair_body(p, carry):
        for b in range(2):
            j = p * 2 + b

            @pl.when(j < trips)
            def _():
                wait_fetch(j, b)

                @pl.when(j + 1 < trips)
                def _():
                    start_fetch(j + 1, 1 - b)

                compute_chunk(b)

        return carry

    lax.fori_loop(0, (BASE_TRIPS + 2) // 2, pair_body, 0)

    pltpu.sync_copy(grid_v, out_hbm.at[wid])


_deposit = functools.partial(
    pl.kernel,
    out_type=jax.ShapeDtypeStruct((NW, GS * GS), jnp.float32),
    mesh=plsc.VectorSubcoreMesh(core_axis_name="c", subcore_axis_name="s"),
    scratch_types=[
        pltpu.VMEM((GS * GS,), jnp.float32),
        pltpu.VMEM((2, BLOCKS, 2, 128), jnp.float32),
        pltpu.VMEM((2, CHUNK), jnp.float32),
        [pltpu.SemaphoreType.DMA, pltpu.SemaphoreType.DMA],
    ],
    compiler_params=pltpu.CompilerParams(needs_layout_passes=False),
)(_deposit_body)


def _sum_partials_body(p_ref, o_ref):
    o_ref[...] = jnp.sum(p_ref[...], axis=0).reshape(GS, GS)


_sum_partials = pl.pallas_call(
    _sum_partials_body,
    out_shape=jax.ShapeDtypeStruct((GS, GS), jnp.float32),
)


def kernel(positions, charges):
    pos_blocked = positions.T.reshape(2, NB, 128).transpose(1, 0, 2)
    partials = _deposit(pos_blocked, charges)
    return _sum_partials(partials)


# speedup vs baseline: 2.3451x; 1.0116x over previous
import functools

import jax
import jax.numpy as jnp
from jax import lax
from jax.experimental import pallas as pl
from jax.experimental.pallas import tpu as pltpu
from jax.experimental.pallas import tpu_sc as plsc

GS = 256
NP = 2_000_000
NC = 2
NS = 16
NW = NC * NS
NB = NP // 128
BLOCKS = 25
CHUNK = BLOCKS * 128
NCHUNK = NP // CHUNK
BASE_TRIPS = NCHUNK // NW
EXTRA = NCHUNK % NW


def _deposit_body(pos_hbm, chg_hbm, out_hbm, grid_v, pos_v, chg_v, sems):
    c = lax.axis_index("c")
    s = lax.axis_index("s")
    wid = s * NC + c
    zeros16 = jnp.zeros((16,), jnp.float32)
    trips = jnp.where(wid < EXTRA, BASE_TRIPS + 1, BASE_TRIPS)

    def start_fetch(j, b):
        ck = wid + j * NW
        pltpu.async_copy(pos_hbm.at[pl.ds(ck * BLOCKS, BLOCKS)],
                         pos_v.at[b], sems[b])
        pltpu.async_copy(chg_hbm.at[pl.ds(ck * CHUNK, CHUNK)],
                         chg_v.at[b], sems[b])

    def wait_fetch(j, b):
        ck = wid + j * NW
        pltpu.make_async_copy(pos_hbm.at[pl.ds(ck * BLOCKS, BLOCKS)],
                              pos_v.at[b], sems[b]).wait()
        pltpu.make_async_copy(chg_hbm.at[pl.ds(ck * CHUNK, CHUNK)],
                              chg_v.at[b], sems[b]).wait()

    def compute_chunk(b):
        @plsc.parallel_loop(0, BLOCKS, unroll=5)
        def block_body(blk):
            for sub in range(8):
                xs = pos_v[b, blk, 0, pl.ds(sub * 16, 16)]
                ys = pos_v[b, blk, 1, pl.ds(sub * 16, 16)]
                cs = chg_v[b, pl.ds(blk * 128 + sub * 16, 16)]
                cxi = xs.astype(jnp.int32)
                cyi = ys.astype(jnp.int32)
                fx = xs - cxi.astype(jnp.float32)
                fy = ys - cyi.astype(jnp.float32)
                cx = jnp.minimum(cxi, GS - 2)
                cy = jnp.minimum(cyi, GS - 2)
                flat = lax.shift_left(cx, 8) + cy
                flat01 = flat + 1
                flat10 = flat + GS
                flat11 = flat + (GS + 1)
                gx = 1.0 - fx
                gy = 1.0 - fy
                a = gx * cs
                b2 = fx * cs
                plsc.addupdate_scatter(grid_v, [flat], a * gy)
                plsc.addupdate_scatter(grid_v, [flat10], b2 * gy)
                plsc.addupdate_scatter(grid_v, [flat01], a * fy)
                plsc.addupdate_scatter(grid_v, [flat11], b2 * fy)

    start_fetch(0, 0)

    @plsc.parallel_loop(0, GS * GS // 16, unroll=8)
    def zero_body(i):
        grid_v[pl.ds(i * 16, 16)] = zeros16

    def pair_body(p, carry):
        for b in range(2):
            j = p * 2 + b

            @pl.when(j < trips)
            def _():
                wait_fetch(j, b)

                @pl.when(j + 1 < trips)
                def _():
                    start_fetch(j + 1, 1 - b)

                compute_chunk(b)

        return carry

    lax.fori_loop(0, (BASE_TRIPS + 2) // 2, pair_body, 0)

    pltpu.sync_copy(grid_v, out_hbm.at[wid])


_deposit = functools.partial(
    pl.kernel,
    out_type=jax.ShapeDtypeStruct((NW, GS * GS), jnp.float32),
    mesh=plsc.VectorSubcoreMesh(core_axis_name="c", subcore_axis_name="s"),
    scratch_types=[
        pltpu.VMEM((GS * GS,), jnp.float32),
        pltpu.VMEM((2, BLOCKS, 2, 128), jnp.float32),
        pltpu.VMEM((2, CHUNK), jnp.float32),
        [pltpu.SemaphoreType.DMA, pltpu.SemaphoreType.DMA],
    ],
    compiler_params=pltpu.CompilerParams(needs_layout_passes=False),
)(_deposit_body)


def _sum_partials_body(p_ref, o_ref):
    o_ref[...] = jnp.sum(p_ref[...], axis=0).reshape(GS, GS)


_sum_partials = pl.pallas_call(
    _sum_partials_body,
    out_shape=jax.ShapeDtypeStruct((GS, GS), jnp.float32),
)


def kernel(positions, charges):
    pos_blocked = positions.T.reshape(2, NB, 128).transpose(1, 0, 2)
    partials = _deposit(pos_blocked, charges)
    return _sum_partials(partials)


# speedup vs baseline: 2.4024x; 1.0245x over previous
import functools

import jax
import jax.numpy as jnp
from jax import lax
from jax.experimental import pallas as pl
from jax.experimental.pallas import tpu as pltpu
from jax.experimental.pallas import tpu_sc as plsc

GS = 256
NP = 2_000_000
NC = 2
NS = 16
NW = NC * NS
NB = NP // 128
BLOCKS = 25
CHUNK = BLOCKS * 128
NCHUNK = NP // CHUNK
BASE_TRIPS = NCHUNK // NW
EXTRA = NCHUNK % NW


def _deposit_body(pos_hbm, chg_hbm, out_hbm, grid_v, pos_v, chg_v, sems):
    c = lax.axis_index("c")
    s = lax.axis_index("s")
    wid = s * NC + c
    zeros16 = jnp.zeros((16,), jnp.float32)
    trips = jnp.where(wid < EXTRA, BASE_TRIPS + 1, BASE_TRIPS)

    def start_fetch(j, b):
        ck = wid + j * NW
        pltpu.async_copy(pos_hbm.at[pl.ds(ck * BLOCKS, BLOCKS)],
                         pos_v.at[b], sems[b])
        pltpu.async_copy(chg_hbm.at[pl.ds(ck * CHUNK, CHUNK)],
                         chg_v.at[b], sems[b])

    def wait_fetch(j, b):
        ck = wid + j * NW
        pltpu.make_async_copy(pos_hbm.at[pl.ds(ck * BLOCKS, BLOCKS)],
                              pos_v.at[b], sems[b]).wait()
        pltpu.make_async_copy(chg_hbm.at[pl.ds(ck * CHUNK, CHUNK)],
                              chg_v.at[b], sems[b]).wait()

    def compute_chunk(b):
        @plsc.parallel_loop(0, BLOCKS, unroll=5)
        def block_body(blk):
            for sub in range(8):
                xs = pos_v[b, blk, 0, pl.ds(sub * 16, 16)]
                ys = pos_v[b, blk, 1, pl.ds(sub * 16, 16)]
                cs = chg_v[b, pl.ds(blk * 128 + sub * 16, 16)]
                cxi = xs.astype(jnp.int32)
                cyi = ys.astype(jnp.int32)
                fx = xs - cxi.astype(jnp.float32)
                fy = ys - cyi.astype(jnp.float32)
                cx = plsc.bitcast(
                    jnp.minimum(plsc.bitcast(cxi, jnp.uint32),
                                jnp.uint32(GS - 2)), jnp.int32)
                cy = plsc.bitcast(
                    jnp.minimum(plsc.bitcast(cyi, jnp.uint32),
                                jnp.uint32(GS - 2)), jnp.int32)
                flat = lax.shift_left(cx, 8) + cy
                flat01 = flat + 1
                flat10 = flat + GS
                flat11 = flat + (GS + 1)
                gx = 1.0 - fx
                gy = 1.0 - fy
                a = gx * cs
                b2 = fx * cs
                plsc.addupdate_scatter(grid_v, [flat], a * gy)
                plsc.addupdate_scatter(grid_v, [flat10], b2 * gy)
                plsc.addupdate_scatter(grid_v, [flat01], a * fy)
                plsc.addupdate_scatter(grid_v, [flat11], b2 * fy)

    start_fetch(0, 0)

    @plsc.parallel_loop(0, GS * GS // 16, unroll=8)
    def zero_body(i):
        grid_v[pl.ds(i * 16, 16)] = zeros16

    def pair_body(p, carry):
        for b in range(2):
            j = p * 2 + b

            @pl.when(j < trips)
            def _():
                wait_fetch(j, b)

                @pl.when(j + 1 < trips)
                def _():
                    start_fetch(j + 1, 1 - b)

                compute_chunk(b)

        return carry

    lax.fori_loop(0, (BASE_TRIPS + 2) // 2, pair_body, 0)

    pltpu.sync_copy(grid_v, out_hbm.at[wid])


_deposit = functools.partial(
    pl.kernel,
    out_type=jax.ShapeDtypeStruct((NW, GS * GS), jnp.float32),
    mesh=plsc.VectorSubcoreMesh(core_axis_name="c", subcore_axis_name="s"),
    scratch_types=[
        pltpu.VMEM((GS * GS,), jnp.float32),
        pltpu.VMEM((2, BLOCKS, 2, 128), jnp.float32),
        pltpu.VMEM((2, CHUNK), jnp.float32),
        [pltpu.SemaphoreType.DMA, pltpu.SemaphoreType.DMA],
    ],
    compiler_params=pltpu.CompilerParams(needs_layout_passes=False),
)(_deposit_body)


def _sum_partials_body(p_ref, o_ref):
    o_ref[...] = jnp.sum(p_ref[...], axis=0).reshape(GS, GS)


_sum_partials = pl.pallas_call(
    _sum_partials_body,
    out_shape=jax.ShapeDtypeStruct((GS, GS), jnp.float32),
)


def kernel(positions, charges):
    pos_blocked = positions.T.reshape(2, NB, 128).transpose(1, 0, 2)
    partials = _deposit(pos_blocked, charges)
    return _sum_partials(partials)
